# baseline (speedup 1.0000x reference)
import functools

import jax
import jax.numpy as jnp
from jax import lax
from jax.experimental import pallas as pl
from jax.experimental.pallas import tpu as pltpu
from jax.experimental.pallas import tpu_sc as plsc

_N = 10000
_E = 160000
_C = 128
_NSPH = 4
_NEL = 10
_AVG_NEIGH = 16.0

_NTILES = 16
_EP = 163840
_EPT = _EP // _NTILES
_BLK = 80
_NBLK = _EPT // _BLK
_NSUP = _NBLK // 4
_WTILES = 10
_RPT = _N // _WTILES
_ZROWS = 40


def _up_body(x_ref, w_ref, o_ref):
    o_ref[...] = jnp.dot(x_ref[...], w_ref[...],
                         preferred_element_type=jnp.float32)


def _linear_up(x, w):
    bn = 1000
    return pl.pallas_call(
        _up_body,
        grid=(_N // bn,),
        in_specs=[
            pl.BlockSpec((bn, _C), lambda i: (i, 0)),
            pl.BlockSpec((_C, _C), lambda i: (0, 0)),
        ],
        out_specs=pl.BlockSpec((bn, _C), lambda i: (i, 0)),
        out_shape=jax.ShapeDtypeStruct((_N, _C), jnp.float32),
    )(x, w)


_sc_mesh = plsc.VectorSubcoreMesh(core_axis_name="c", subcore_axis_name="s")


@functools.partial(
    pl.kernel,
    mesh=_sc_mesh,
    out_type=jax.ShapeDtypeStruct((_NSPH, _N, _C), jnp.float32),
    scratch_types=[
        pltpu.VMEM_SHARED((_N, _C), jnp.float32),
        pltpu.VMEM((_BLK,), jnp.int32),
        pltpu.VMEM((_BLK,), jnp.int32),
        pltpu.VMEM((_BLK,), jnp.int32),
        pltpu.VMEM((_BLK,), jnp.float32),
        pltpu.VMEM((_BLK, _C), jnp.float32),
        pltpu.VMEM((_BLK, _C), jnp.float32),
        pltpu.VMEM((_ZROWS, _C), jnp.float32),
        pltpu.SemaphoreType.DMA,
        pltpu.SemaphoreType.DMA,
        pltpu.SemaphoreType.DMA,
        pltpu.SemaphoreType.DMA,
        pltpu.SemaphoreType.DMA,
    ],
)
def _sc_messages(h_hbm, ef2_hbm, attrs_t_hbm, snd_hbm, rcv_hbm, out_hbm,
                 acc, snd_v, rcv_v, eidx_v, attr_v, ef_v, h_v, zbuf,
                 sem_a, sem_b, sem_c, sem_d, sem_e):
    core = lax.axis_index("c")
    sid = lax.axis_index("s")
    tile_e0 = sid * _EPT
    row0 = sid * _RPT

    def _zrow(i, carry):
        for j in range(_C // 16):
            zbuf[i, pl.ds(j * 16, 16)] = jnp.zeros((16,), jnp.float32)
        return carry

    lax.fori_loop(0, _ZROWS, _zrow, 0)

    def _round(r, round_carry):
        m = core * 2 + r
        lval = jnp.minimum(m, 1).astype(jnp.int32)

        @pl.when(sid < _WTILES)
        def _zero():
            for k in range(_RPT // _ZROWS):
                pltpu.async_copy(
                    zbuf, acc.at[pl.ds(row0 + k * _ZROWS, _ZROWS)], sem_a)
            for k in range(_RPT // _ZROWS):
                pltpu.make_async_copy(
                    zbuf, acc.at[pl.ds(row0 + k * _ZROWS, _ZROWS)],
                    sem_a).wait()

        plsc.subcore_barrier()

        def _block(b, carry):
            e0 = tile_e0 + b * _BLK
            cp_s = pltpu.async_copy(snd_hbm.at[pl.ds(e0, _BLK)], snd_v, sem_a)
            cp_r = pltpu.async_copy(rcv_hbm.at[pl.ds(e0, _BLK)], rcv_v, sem_b)
            cp_t = pltpu.async_copy(
                attrs_t_hbm.at[pl.ds(m * _EP + e0, _BLK)], attr_v, sem_c)
            for j in range(_BLK // 16):
                ii = lax.iota(jnp.int32, 16)
                eidx_v[pl.ds(j * 16, 16)] = 2 * (e0 + j * 16 + ii) + lval
            cp_e = pltpu.async_copy(ef2_hbm.at[eidx_v], ef_v, sem_d)
            cp_s.wait()
            cp_h = pltpu.async_copy(h_hbm.at[snd_v], h_v, sem_e)
            cp_r.wait()
            cp_t.wait()
            cp_e.wait()
            cp_h.wait()

            def _group(g, inner):
                av = attr_v[pl.ds(g * 16, 16)]
                for k in range(16):
                    bc = lax.gather(
                        av, jnp.full((16, 1), k, jnp.int32),
                        lax.GatherDimensionNumbers(
                            offset_dims=(), collapsed_slice_dims=(0,),
                            start_index_map=(0,)),
                        slice_sizes=(1,),
                        mode=lax.GatherScatterMode.PROMISE_IN_BOUNDS)
                    e = g * 16 + k
                    for j in range(_C // 16):
                        sl = pl.ds(j * 16, 16)
                        h_v[e, sl] = bc * ef_v[e, sl] * h_v[e, sl]
                return inner

            lax.fori_loop(0, _BLK // 16, _group, 0)
            pltpu.sync_copy(h_v, acc.at[rcv_v], add=True)
            return carry

        lax.fori_loop(0, _NBLK, _block, 0)
        plsc.subcore_barrier()

        @pl.when(sid < _WTILES)
        def _writeout():
            pltpu.sync_copy(acc.at[pl.ds(row0, _RPT)],
                            out_hbm.at[m, pl.ds(row0, _RPT)])

        plsc.subcore_barrier()
        return round_carry

    lax.fori_loop(0, 2, _round, 0)


def _post_body(attrs_ref, msg_ref, wlin_ref, wskip_ref, out_ref):
    bn = attrs_ref.shape[0]
    na = attrs_ref[...]
    mx = jnp.max(na, axis=-1, keepdims=True)
    iot = lax.broadcasted_iota(jnp.int32, (bn, _NEL), 1)
    idxs = jnp.where(na >= mx, iot, _NEL)
    elem = jnp.min(idxs, axis=-1, keepdims=True)
    inv = jnp.float32(1.0 / _AVG_NEIGH)
    for mm in range(_NSPH):
        lv = 0 if mm == 0 else 1
        mixed = jnp.dot(msg_ref[mm], wlin_ref[lv],
                        preferred_element_type=jnp.float32) * inv
        acc = jnp.zeros((bn, _C), jnp.float32)
        for el in range(_NEL):
            msk = (elem == el).astype(jnp.float32)
            acc = acc + jnp.dot(mixed * msk, wskip_ref[el, lv],
                                preferred_element_type=jnp.float32)
        out_ref[:, mm, :] = acc


def _post(node_attrs, msg, w_lin, w_skip):
    bn = 1000
    return pl.pallas_call(
        _post_body,
        grid=(_N // bn,),
        in_specs=[
            pl.BlockSpec((bn, _NEL), lambda i: (i, 0)),
            pl.BlockSpec((_NSPH, bn, _C), lambda i: (0, i, 0)),
            pl.BlockSpec((2, _C, _C), lambda i: (0, 0, 0)),
            pl.BlockSpec((_NEL, 2, _C, _C), lambda i: (0, 0, 0, 0)),
        ],
        out_specs=pl.BlockSpec((bn, _NSPH, _C), lambda i: (i, 0, 0)),
        out_shape=jax.ShapeDtypeStruct((_N, _NSPH, _C), jnp.float32),
    )(node_attrs, msg, w_lin, w_skip)


def kernel(node_attrs, node_feats, edge_attrs, edge_feats, W_up, W_lin,
           W_skip, edge_index):
    pad = _EP - _E
    sender = jnp.pad(edge_index[1], (0, pad))
    receiver = jnp.pad(edge_index[0], (0, pad))
    ef2 = jnp.pad(edge_feats, ((0, pad), (0, 0))).reshape(2 * _EP, _C)
    attrs_t = jnp.pad(edge_attrs, ((0, pad), (0, 0))).T.reshape(-1)
    h = _linear_up(node_feats, W_up)
    msg = _sc_messages(h, ef2, attrs_t, sender, receiver)
    return _post(node_attrs, msg, W_lin, W_skip)


# speedup vs baseline: 1.4167x; 1.4167x over previous
import functools

import jax
import jax.numpy as jnp
from jax import lax
from jax.experimental import pallas as pl
from jax.experimental.pallas import tpu as pltpu
from jax.experimental.pallas import tpu_sc as plsc

_N = 10000
_E = 160000
_C = 128
_NSPH = 4
_NEL = 10
_AVG_NEIGH = 16.0

_NTILES = 16
_EPT = _E // _NTILES
_BLK = 80
_NBLK = _EPT // _BLK
_WTILES = 10
_RPT = _N // _WTILES
_ZROWS = 40


def _up_body(x_ref, w_ref, o_ref):
    o_ref[...] = jnp.dot(x_ref[...], w_ref[...],
                         preferred_element_type=jnp.float32)


def _linear_up(x, w):
    bn = 1000
    return pl.pallas_call(
        _up_body,
        grid=(_N // bn,),
        in_specs=[
            pl.BlockSpec((bn, _C), lambda i: (i, 0)),
            pl.BlockSpec((_C, _C), lambda i: (0, 0)),
        ],
        out_specs=pl.BlockSpec((bn, _C), lambda i: (i, 0)),
        out_shape=jax.ShapeDtypeStruct((_N, _C), jnp.float32),
    )(x, w)


_sc_mesh = plsc.VectorSubcoreMesh(core_axis_name="c", subcore_axis_name="s")


@functools.partial(
    pl.kernel,
    mesh=_sc_mesh,
    out_type=jax.ShapeDtypeStruct((_NSPH, _N, _C), jnp.float32),
    scratch_types=[
        pltpu.VMEM_SHARED((_N, _C), jnp.float32),
        pltpu.VMEM((_BLK,), jnp.int32),
        pltpu.VMEM((_BLK,), jnp.int32),
        pltpu.VMEM((_BLK,), jnp.int32),
        pltpu.VMEM((_BLK,), jnp.float32),
        pltpu.VMEM((_BLK, _C), jnp.float32),
        pltpu.VMEM((_BLK, _C), jnp.float32),
        pltpu.VMEM((_ZROWS, _C), jnp.float32),
        pltpu.SemaphoreType.DMA,
        pltpu.SemaphoreType.DMA,
        pltpu.SemaphoreType.DMA,
        pltpu.SemaphoreType.DMA,
        pltpu.SemaphoreType.DMA,
    ],
)
def _sc_messages(h_hbm, ef2_hbm, attrs_t_hbm, snd_hbm, rcv_hbm, out_hbm,
                 acc, snd_v, rcv_v, eidx_v, attr_v, ef_v, h_v, zbuf,
                 sem_a, sem_b, sem_c, sem_d, sem_e):
    core = lax.axis_index("c")
    sid = lax.axis_index("s")
    tile_e0 = sid * _EPT
    row0 = sid * _RPT

    def _zrow(i, carry):
        for j in range(_C // 16):
            zbuf[i, pl.ds(j * 16, 16)] = jnp.zeros((16,), jnp.float32)
        return carry

    lax.fori_loop(0, _ZROWS, _zrow, 0)

    for r in range(2):
        m = core * 2 + r
        lval = jnp.minimum(m, 1).astype(jnp.int32)

        @pl.when(sid < _WTILES)
        def _zero():
            for k in range(_RPT // _ZROWS):
                pltpu.sync_copy(zbuf,
                                acc.at[pl.ds(row0 + k * _ZROWS, _ZROWS)])

        plsc.subcore_barrier()

        def _block(b, carry):
            e0 = tile_e0 + b * _BLK
            cp_s = pltpu.async_copy(snd_hbm.at[pl.ds(e0, _BLK)], snd_v, sem_a)
            cp_r = pltpu.async_copy(rcv_hbm.at[pl.ds(e0, _BLK)], rcv_v, sem_b)
            cp_t = pltpu.async_copy(
                attrs_t_hbm.at[pl.ds(m * _E + e0, _BLK)], attr_v, sem_c)
            for j in range(_BLK // 16):
                ii = lax.iota(jnp.int32, 16)
                eidx_v[pl.ds(j * 16, 16)] = 2 * (e0 + j * 16 + ii) + lval
            cp_e = pltpu.async_copy(ef2_hbm.at[eidx_v], ef_v, sem_d)
            cp_s.wait()
            cp_h = pltpu.async_copy(h_hbm.at[snd_v], h_v, sem_e)
            cp_r.wait()
            cp_t.wait()
            cp_e.wait()
            cp_h.wait()

            def _group(g, inner):
                av = attr_v[pl.ds(g * 16, 16)]
                for k in range(16):
                    bc = lax.gather(
                        av, jnp.full((16, 1), k, jnp.int32),
                        lax.GatherDimensionNumbers(
                            offset_dims=(), collapsed_slice_dims=(0,),
                            start_index_map=(0,)),
                        slice_sizes=(1,),
                        mode=lax.GatherScatterMode.PROMISE_IN_BOUNDS)
                    e = g * 16 + k
                    for j in range(_C // 16):
                        sl = pl.ds(j * 16, 16)
                        h_v[e, sl] = bc * ef_v[e, sl] * h_v[e, sl]
                return inner

            lax.fori_loop(0, _BLK // 16, _group, 0)
            pltpu.sync_copy(h_v, acc.at[rcv_v], add=True)
            return carry

        lax.fori_loop(0, _NBLK, _block, 0)
        plsc.subcore_barrier()

        @pl.when(sid < _WTILES)
        def _writeout():
            pltpu.sync_copy(acc.at[pl.ds(row0, _RPT)],
                            out_hbm.at[m, pl.ds(row0, _RPT)])

        plsc.subcore_barrier()


def _post_body(attrs_ref, msg_ref, wlin_ref, wskip_ref, out_ref):
    bn = attrs_ref.shape[0]
    na = attrs_ref[...]
    mx = jnp.max(na, axis=-1, keepdims=True)
    iot = lax.broadcasted_iota(jnp.int32, (bn, _NEL), 1)
    idxs = jnp.where(na >= mx, iot, _NEL)
    elem = jnp.min(idxs, axis=-1, keepdims=True)
    inv = jnp.float32(1.0 / _AVG_NEIGH)
    for mm in range(_NSPH):
        lv = 0 if mm == 0 else 1
        mixed = jnp.dot(msg_ref[mm], wlin_ref[lv],
                        preferred_element_type=jnp.float32) * inv
        acc = jnp.zeros((bn, _C), jnp.float32)
        for el in range(_NEL):
            msk = (elem == el).astype(jnp.float32)
            acc = acc + jnp.dot(mixed * msk, wskip_ref[el, lv],
                                preferred_element_type=jnp.float32)
        out_ref[:, mm, :] = acc


def _post(node_attrs, msg, w_lin, w_skip):
    bn = 1000
    return pl.pallas_call(
        _post_body,
        grid=(_N // bn,),
        in_specs=[
            pl.BlockSpec((bn, _NEL), lambda i: (i, 0)),
            pl.BlockSpec((_NSPH, bn, _C), lambda i: (0, i, 0)),
            pl.BlockSpec((2, _C, _C), lambda i: (0, 0, 0)),
            pl.BlockSpec((_NEL, 2, _C, _C), lambda i: (0, 0, 0, 0)),
        ],
        out_specs=pl.BlockSpec((bn, _NSPH, _C), lambda i: (i, 0, 0)),
        out_shape=jax.ShapeDtypeStruct((_N, _NSPH, _C), jnp.float32),
    )(node_attrs, msg, w_lin, w_skip)


def kernel(node_attrs, node_feats, edge_attrs, edge_feats, W_up, W_lin,
           W_skip, edge_index):
    sender = edge_index[1]
    receiver = edge_index[0]
    ef2 = edge_feats.reshape(2 * _E, _C)
    attrs_t = edge_attrs.T.reshape(-1)
    h = _linear_up(node_feats, W_up)
    msg = _sc_messages(h, ef2, attrs_t, sender, receiver)
    return _post(node_attrs, msg, W_lin, W_skip)


# speedup vs baseline: 1.5174x; 1.0711x over previous
import functools

import jax
import jax.numpy as jnp
from jax import lax
from jax.experimental import pallas as pl
from jax.experimental.pallas import tpu as pltpu
from jax.experimental.pallas import tpu_sc as plsc

_N = 10000
_E = 160000
_C = 128
_NSPH = 4
_NEL = 10
_AVG_NEIGH = 16.0

_NTILES = 16
_EPT = _E // _NTILES
_BLK = 80
_NBLK = _EPT // _BLK
_WTILES = 10
_RPT = _N // _WTILES
_ZROWS = 40


def _up_body(x_ref, w_ref, o_ref):
    o_ref[...] = jnp.dot(x_ref[...], w_ref[...],
                         preferred_element_type=jnp.float32)


def _linear_up(x, w):
    bn = 1000
    return pl.pallas_call(
        _up_body,
        grid=(_N // bn,),
        in_specs=[
            pl.BlockSpec((bn, _C), lambda i: (i, 0)),
            pl.BlockSpec((_C, _C), lambda i: (0, 0)),
        ],
        out_specs=pl.BlockSpec((bn, _C), lambda i: (i, 0)),
        out_shape=jax.ShapeDtypeStruct((_N, _C), jnp.float32),
    )(x, w)


_sc_mesh = plsc.VectorSubcoreMesh(core_axis_name="c", subcore_axis_name="s")


@functools.partial(
    pl.kernel,
    mesh=_sc_mesh,
    out_type=jax.ShapeDtypeStruct((_NSPH, _N, _C), jnp.float32),
    scratch_types=[
        pltpu.VMEM_SHARED((_N, _C), jnp.float32),
        pltpu.VMEM((_BLK,), jnp.int32),
        pltpu.VMEM((2, _BLK), jnp.int32),
        pltpu.VMEM((_BLK,), jnp.int32),
        pltpu.VMEM((_BLK,), jnp.float32),
        pltpu.VMEM((_BLK, _C), jnp.float32),
        pltpu.VMEM((2, _BLK, _C), jnp.float32),
        pltpu.VMEM((_ZROWS, _C), jnp.float32),
        pltpu.SemaphoreType.DMA,
        pltpu.SemaphoreType.DMA,
        pltpu.SemaphoreType.DMA,
        pltpu.SemaphoreType.DMA,
        pltpu.SemaphoreType.DMA,
        [pltpu.SemaphoreType.DMA] * 2,
    ],
)
def _sc_messages(h_hbm, ef2_hbm, attrs_t_hbm, snd_hbm, rcv_hbm, out_hbm,
                 acc, snd_v, rcv_v, eidx_v, attr_v, ef_v, h_v, zbuf,
                 sem_a, sem_b, sem_c, sem_d, sem_e, sem_sc):
    core = lax.axis_index("c")
    sid = lax.axis_index("s")
    tile_e0 = sid * _EPT
    row0 = sid * _RPT

    def _zrow(i, carry):
        for j in range(_C // 16):
            zbuf[i, pl.ds(j * 16, 16)] = jnp.zeros((16,), jnp.float32)
        return carry

    lax.fori_loop(0, _ZROWS, _zrow, 0)

    for r in range(2):
        m = core * 2 + r
        lval = jnp.minimum(m, 1).astype(jnp.int32)

        @pl.when(sid < _WTILES)
        def _zero():
            for k in range(_RPT // _ZROWS):
                pltpu.sync_copy(zbuf,
                                acc.at[pl.ds(row0 + k * _ZROWS, _ZROWS)])

        plsc.subcore_barrier()

        def _do_block(b, s, wait_prev):
            e0 = tile_e0 + b * _BLK
            cp_s = pltpu.async_copy(snd_hbm.at[pl.ds(e0, _BLK)], snd_v, sem_a)
            cp_r = pltpu.async_copy(rcv_hbm.at[pl.ds(e0, _BLK)],
                                    rcv_v.at[s], sem_b)
            cp_t = pltpu.async_copy(
                attrs_t_hbm.at[pl.ds(m * _E + e0, _BLK)], attr_v, sem_c)
            for j in range(_BLK // 16):
                ii = lax.iota(jnp.int32, 16)
                eidx_v[pl.ds(j * 16, 16)] = 2 * (e0 + j * 16 + ii) + lval
            cp_e = pltpu.async_copy(ef2_hbm.at[eidx_v], ef_v, sem_d)
            cp_s.wait()
            cp_h = pltpu.async_copy(h_hbm.at[snd_v], h_v.at[s], sem_e)
            cp_r.wait()
            cp_t.wait()
            cp_e.wait()
            cp_h.wait()
            wait_prev()

            def _group(g, inner):
                av = attr_v[pl.ds(g * 16, 16)]
                for k in range(16):
                    bc = lax.gather(
                        av, jnp.full((16, 1), k, jnp.int32),
                        lax.GatherDimensionNumbers(
                            offset_dims=(), collapsed_slice_dims=(0,),
                            start_index_map=(0,)),
                        slice_sizes=(1,),
                        mode=lax.GatherScatterMode.PROMISE_IN_BOUNDS)
                    e = g * 16 + k
                    for j in range(_C // 16):
                        sl = pl.ds(j * 16, 16)
                        h_v[s, e, sl] = bc * ef_v[e, sl] * h_v[s, e, sl]
                return inner

            lax.fori_loop(0, _BLK // 16, _group, 0)
            pltpu.async_copy(h_v.at[s], acc.at[rcv_v.at[s]], sem_sc[s],
                             add=True)

        def _sc_wait(s):
            pltpu.make_async_copy(h_v.at[s], acc.at[rcv_v.at[s]],
                                  sem_sc[s]).wait()

        def _pair(t, carry):
            def _w0():
                @pl.when(t > 0)
                def _():
                    _sc_wait(1)

            _do_block(2 * t, 0, _w0)
            _do_block(2 * t + 1, 1, lambda: _sc_wait(0))
            return carry

        lax.fori_loop(0, (_NBLK - 1) // 2, _pair, 0)
        _do_block(_NBLK - 1, 0, lambda: _sc_wait(1))
        _sc_wait(0)
        plsc.subcore_barrier()

        @pl.when(sid < _WTILES)
        def _writeout():
            pltpu.sync_copy(acc.at[pl.ds(row0, _RPT)],
                            out_hbm.at[m, pl.ds(row0, _RPT)])

        plsc.subcore_barrier()


def _post_body(attrs_ref, msg_ref, wlin_ref, wskip_ref, out_ref):
    bn = attrs_ref.shape[0]
    na = attrs_ref[...]
    mx = jnp.max(na, axis=-1, keepdims=True)
    iot = lax.broadcasted_iota(jnp.int32, (bn, _NEL), 1)
    idxs = jnp.where(na >= mx, iot, _NEL)
    elem = jnp.min(idxs, axis=-1, keepdims=True)
    inv = jnp.float32(1.0 / _AVG_NEIGH)
    for mm in range(_NSPH):
        lv = 0 if mm == 0 else 1
        mixed = jnp.dot(msg_ref[mm], wlin_ref[lv],
                        preferred_element_type=jnp.float32) * inv
        acc = jnp.zeros((bn, _C), jnp.float32)
        for el in range(_NEL):
            msk = (elem == el).astype(jnp.float32)
            acc = acc + jnp.dot(mixed * msk, wskip_ref[el, lv],
                                preferred_element_type=jnp.float32)
        out_ref[:, mm, :] = acc


def _post(node_attrs, msg, w_lin, w_skip):
    bn = 1000
    return pl.pallas_call(
        _post_body,
        grid=(_N // bn,),
        in_specs=[
            pl.BlockSpec((bn, _NEL), lambda i: (i, 0)),
            pl.BlockSpec((_NSPH, bn, _C), lambda i: (0, i, 0)),
            pl.BlockSpec((2, _C, _C), lambda i: (0, 0, 0)),
            pl.BlockSpec((_NEL, 2, _C, _C), lambda i: (0, 0, 0, 0)),
        ],
        out_specs=pl.BlockSpec((bn, _NSPH, _C), lambda i: (i, 0, 0)),
        out_shape=jax.ShapeDtypeStruct((_N, _NSPH, _C), jnp.float32),
    )(node_attrs, msg, w_lin, w_skip)


def kernel(node_attrs, node_feats, edge_attrs, edge_feats, W_up, W_lin,
           W_skip, edge_index):
    sender = edge_index[1]
    receiver = edge_index[0]
    ef2 = edge_feats.reshape(2 * _E, _C)
    attrs_t = edge_attrs.T.reshape(-1)
    h = _linear_up(node_feats, W_up)
    msg = _sc_messages(h, ef2, attrs_t, sender, receiver)
    return _post(node_attrs, msg, W_lin, W_skip)
